# Initial kernel scaffold; baseline (speedup 1.0000x reference)
import jax
import jax.numpy as jnp
from jax.experimental import pallas as pl


def kernel(softmax):
    raise NotImplementedError("write your pallas kernel here")


# speedup vs baseline: 2.7363x; 2.7363x over previous
import numpy as np
import jax
import jax.numpy as jnp
from jax.experimental import pallas as pl
from jax.experimental.pallas import tpu as pltpu

TOPK = 4
BLK = 8192
BIGI = np.int32(2**30)
NEG = np.float32(-np.inf)
TINY = np.float32(np.finfo(np.float32).tiny)


def _select_topk(vals, idx, k=TOPK):
    out_v, out_i = [], []
    for _ in range(k):
        m = jnp.max(vals, axis=1, keepdims=True)
        sel = jnp.min(jnp.where(vals == m, idx, BIGI), axis=1, keepdims=True)
        out_v.append(m)
        out_i.append(sel)
        kill = (vals == m) & (idx == sel)
        vals = jnp.where(kill, NEG, vals)
    return jnp.concatenate(out_v, axis=1), jnp.concatenate(out_i, axis=1)


def _threefry_bits(p_u32):
    k0 = jnp.uint32(0)
    k1 = jnp.uint32(42)
    k2 = k0 ^ k1 ^ jnp.uint32(0x1BD11BDA)
    ks = [k0, k1, k2]
    rot0 = (13, 15, 26, 6)
    rot1 = (17, 29, 16, 24)

    def rotl(x, d):
        return (x << jnp.uint32(d)) | (x >> jnp.uint32(32 - d))

    x0 = jnp.zeros_like(p_u32) + k0
    x1 = p_u32 + k1
    for r in range(5):
        for d in (rot0 if r % 2 == 0 else rot1):
            x0 = x0 + x1
            x1 = rotl(x1, d) ^ x0
        x0 = x0 + ks[(r + 1) % 3]
        x1 = x1 + ks[(r + 2) % 3] + jnp.uint32(r + 1)
    return x0 ^ x1


def _gumbel_at(p_i32):
    bits = _threefry_bits(p_i32.astype(jnp.uint32))
    fb = (bits >> jnp.uint32(9)) | jnp.uint32(0x3F800000)
    f = jax.lax.bitcast_convert_type(fb, jnp.float32) - jnp.float32(1.0)
    u = jnp.maximum(TINY, f + TINY)
    return -jnp.log(-jnp.log(u))


def _make_kernel(B, N, nblk):
    def body(x_ref, out_ref, s_ref, sv, si, srv):
        ph = pl.program_id(0)
        step = pl.program_id(1)

        @pl.when((ph == 0) & (step == 0))
        def _init():
            sv[...] = jnp.full((B, TOPK), NEG, jnp.float32)
            si[...] = jnp.full((B, TOPK), BIGI, jnp.int32)

        col = jax.lax.broadcasted_iota(jnp.int32, (B, BLK), 1) + step * BLK

        @pl.when(ph == 0)
        def _scan():
            x = jnp.where(col < N, x_ref[...], NEG)
            bv, bi = _select_topk(x, col)
            cv = jnp.concatenate([sv[...], bv], axis=1)
            ci = jnp.concatenate([si[...], bi], axis=1)
            nv, ni = _select_topk(cv, ci)
            sv[...] = nv
            si[...] = ni

            @pl.when(step == nblk - 1)
            def _finalize():
                v = sv[...]
                ix = si[...]
                rv = v / jnp.sum(v, axis=1, keepdims=True)
                srv[...] = rv
                p = ix + jax.lax.broadcasted_iota(jnp.int32, (B, TOPK), 0) * N
                score = jnp.log(rv + jnp.float32(1e-20)) + _gumbel_at(p)
                m = jnp.max(score, axis=1, keepdims=True)
                j = jax.lax.broadcasted_iota(jnp.int32, (B, TOPK), 1)
                jsel = jnp.min(
                    jnp.where(score == m, j, BIGI), axis=1, keepdims=True
                )
                s_ref[...] = jnp.sum(
                    jnp.where(j == jsel, ix, 0), axis=1, keepdims=True
                )

        @pl.when(ph == 1)
        def _write():
            hit = jnp.any(si[...] // BLK == step)

            @pl.when(hit)
            def _scatter_block():
                acc = jnp.zeros((B, BLK), jnp.float32)
                for jj in range(TOPK):
                    cj = si[:, pl.ds(jj, 1)]
                    vj = srv[:, pl.ds(jj, 1)]
                    acc = jnp.where(col == cj, vj, acc)
                out_ref[...] = acc

            @pl.when(jnp.logical_not(hit))
            def _zeros_block():
                out_ref[...] = jnp.zeros((B, BLK), jnp.float32)

    return body


def kernel(softmax):
    B, N = softmax.shape
    nblk = (N + BLK - 1) // BLK

    renorm, s2d = pl.pallas_call(
        _make_kernel(B, N, nblk),
        grid=(2, nblk),
        in_specs=[pl.BlockSpec((B, BLK), lambda ph, i: (0, i * (1 - ph)))],
        out_specs=[
            pl.BlockSpec((B, BLK), lambda ph, i: (0, i * ph)),
            pl.BlockSpec((B, 1), lambda ph, i: (0, 0)),
        ],
        out_shape=[
            jax.ShapeDtypeStruct((B, N), jnp.float32),
            jax.ShapeDtypeStruct((B, 1), jnp.int32),
        ],
        scratch_shapes=[
            pltpu.VMEM((B, TOPK), jnp.float32),
            pltpu.VMEM((B, TOPK), jnp.int32),
            pltpu.VMEM((B, TOPK), jnp.float32),
        ],
    )(softmax)

    return renorm, s2d.reshape(B)
